# baseline (speedup 1.0000x reference)
import functools

import jax
import jax.numpy as jnp
from jax import lax
from jax.experimental import pallas as pl
from jax.experimental import layout as jlayout
from jax.experimental.pallas import tpu as pltpu
from jax.experimental.pallas import tpu_sc as plsc

KMER_SIZE = 4
BATCH = 1024
SEQ_LEN = 512
EMBED_DIM = 64

NUM_CORES = 2
NUM_SUBCORES = 16
NUM_WORKERS = NUM_CORES * NUM_SUBCORES
SEQS_PER_WORKER = BATCH // NUM_WORKERS
PER_WORKER = SEQS_PER_WORKER * SEQ_LEN
CHUNK = SEQ_LEN
N_CHUNKS = PER_WORKER // CHUNK


def _sc_body(idx_hbm, table_hbm, out_hbm, idx_v, rows_v, g0, g1, s0, s1):
    gsems = (g0, g1)
    ssems = (s0, s1)
    wid = lax.axis_index("s") * NUM_CORES + lax.axis_index("c")
    seq_base = wid * SEQS_PER_WORKER
    pltpu.sync_copy(idx_hbm.at[pl.ds(seq_base, N_CHUNKS)], idx_v)

    def fire_gather(c, slot):
        pltpu.async_copy(table_hbm.at[idx_v.at[c]], rows_v.at[slot], gsems[slot])

    def wait_gather(slot):
        pltpu.make_async_copy(
            table_hbm.at[idx_v.at[0]], rows_v.at[slot], gsems[slot]
        ).wait()

    def fire_scatter(c, slot):
        pltpu.async_copy(rows_v.at[slot], out_hbm.at[seq_base + c], ssems[slot])

    def wait_scatter(slot):
        pltpu.make_async_copy(
            rows_v.at[slot], out_hbm.at[seq_base], ssems[slot]
        ).wait()

    def mask(slot):
        zeros = jnp.zeros((16,), jnp.float32)
        for r in range(KMER_SIZE - 1):
            for l in range(EMBED_DIM // 16):
                rows_v[slot, r, pl.ds(l * 16, 16)] = zeros

    fire_gather(0, 0)
    wait_gather(0)
    mask(0)
    fire_scatter(0, 0)
    fire_gather(1, 1)

    def group(g, _):
        for b in range(2):
            c = 2 * g + 1 + b
            slot = (1 + b) % 2
            wait_gather(slot)
            mask(slot)
            fire_scatter(c, slot)
            wait_scatter(1 - slot)
            fire_gather(c + 1, 1 - slot)
        return 0

    lax.fori_loop(0, (N_CHUNKS - 2) // 2, group, 0)

    wait_gather(1)
    mask(1)
    fire_scatter(N_CHUNKS - 1, 1)
    wait_scatter(0)
    wait_scatter(1)


@functools.lru_cache(maxsize=1)
def _out_format():
    return jlayout.Format(
        jlayout.Layout(major_to_minor=(0, 1, 2), tiling=()),
        jax.sharding.SingleDeviceSharding(jax.devices()[0]),
    )


@jax.jit
def _encode(kmer_indices, kmer_table):
    mesh = plsc.VectorSubcoreMesh(
        core_axis_name="c",
        subcore_axis_name="s",
        num_cores=NUM_CORES,
        num_subcores=NUM_SUBCORES,
    )
    run = pl.kernel(
        _sc_body,
        out_type=jax.ShapeDtypeStruct((BATCH, SEQ_LEN, EMBED_DIM), jnp.float32),
        mesh=mesh,
        scratch_types=[
            pltpu.VMEM((N_CHUNKS, CHUNK), jnp.int32),
            pltpu.VMEM((2, CHUNK, EMBED_DIM), jnp.float32),
            pltpu.SemaphoreType.DMA,
            pltpu.SemaphoreType.DMA,
            pltpu.SemaphoreType.DMA,
            pltpu.SemaphoreType.DMA,
        ],
        compiler_params=pltpu.CompilerParams(use_tc_tiling_on_sc=False),
    )
    return run(kmer_indices, kmer_table)


def kernel(kmer_indices, kmer_table):
    return jax.jit(_encode, out_shardings=_out_format())(
        kmer_indices, kmer_table
    )
